# baseline (speedup 1.0000x reference)
import functools

import jax
import jax.numpy as jnp
from jax import lax
from jax.experimental import pallas as pl
from jax.experimental.pallas import tpu as pltpu
from jax.experimental.pallas import tpu_sc as plsc

B, T, D = 2, 2048, 1024
DFF = 4096
E = 8
K = 2
N = B * T
A = N * K
BLK = 512
NB = A // BLK + E
NP = NB * BLK
NBC = (NB + 15) // 16
DFC = 2048
DC = DFF // DFC

NW = 32
TPW = N // NW
CH = TPW // 16

_mesh = functools.partial(
    plsc.VectorSubcoreMesh, core_axis_name="c", subcore_axis_name="s")
_SC_PARAMS = pltpu.CompilerParams(needs_layout_passes=False)


def _wid():
    return lax.axis_index("s") * 2 + lax.axis_index("c")


def _router_body(x_ref, wr_ref, out_ref):
    logits = jnp.dot(x_ref[...], wr_ref[...], preferred_element_type=jnp.float32)
    lane = lax.broadcasted_iota(jnp.int32, logits.shape, 1)
    neg = jnp.float32(-1e30)
    l0 = jnp.where(lane < E, logits, neg)
    m0 = jnp.max(l0, axis=1)
    i0 = jnp.min(jnp.where(l0 >= m0[:, None], lane, 127), axis=1)
    l1 = jnp.where(lane == i0[:, None], neg, l0)
    m1 = jnp.max(l1, axis=1)
    i1 = jnp.min(jnp.where(l1 >= m1[:, None], lane, 127), axis=1)
    w0 = 1.0 / (1.0 + jnp.exp(m1 - m0))
    w1 = 1.0 - w0
    out_ref[0:1, :] = i0.astype(jnp.float32)[None, :]
    out_ref[1:2, :] = i1.astype(jnp.float32)[None, :]
    out_ref[2:3, :] = w0[None, :]
    out_ref[3:4, :] = w1[None, :]


def _route(x_flat, Wr):
    wr_pad = jnp.pad(Wr, ((0, 0), (0, 128 - E)))
    r = pl.pallas_call(
        _router_body,
        out_shape=jax.ShapeDtypeStruct((8, N), jnp.float32),
    )(x_flat, wr_pad)
    i0 = r[0].astype(jnp.int32)
    i1 = r[1].astype(jnp.int32)
    return i0, i1, r[2], r[3]


def _count_body(i0_hbm, i1_hbm, lrA_hbm, lrB_hbm, cnt_hbm, eid_v, lr_v, cnt_ref):
    w = _wid()
    lane = lax.iota(jnp.int32, 16)
    cnt_ref[...] = jnp.zeros((16,), jnp.int32)

    for half, (src, dst) in enumerate(((i0_hbm, lrA_hbm), (i1_hbm, lrB_hbm))):
        pltpu.sync_copy(src.at[pl.ds(w * TPW, TPW)], eid_v)

        def chunk(j, _):
            v = eid_v[pl.ds(j * 16, 16)]
            prior = plsc.load_gather(cnt_ref, [v])
            within = jnp.zeros((16,), jnp.int32)
            cnt_vec = cnt_ref[...]
            for e in range(E):
                m = v == e
                cs = plsc.cumsum(jnp.where(m, 1, 0).astype(jnp.int32))
                within = jnp.where(m, cs - 1, within)
                tot_e = jnp.sum(jnp.where(m, 1, 0).astype(jnp.int32))
                cnt_vec = jnp.where(lane == e, cnt_vec + tot_e, cnt_vec)
            cnt_ref[...] = cnt_vec
            lr_v[pl.ds(j * 16, 16)] = prior + within
            return 0

        lax.fori_loop(0, CH, chunk, 0)
        pltpu.sync_copy(lr_v, dst.at[pl.ds(w * TPW, TPW)])

    pltpu.sync_copy(cnt_ref, cnt_hbm.at[w])


def _sc_count(i0, i1):
    f = pl.kernel(
        _count_body,
        out_type=[
            jax.ShapeDtypeStruct((N,), jnp.int32),
            jax.ShapeDtypeStruct((N,), jnp.int32),
            jax.ShapeDtypeStruct((NW, 16), jnp.int32),
        ],
        mesh=_mesh(),
        compiler_params=_SC_PARAMS,
        scratch_types=[
            pltpu.VMEM((TPW,), jnp.int32),
            pltpu.VMEM((TPW,), jnp.int32),
            pltpu.VMEM((16,), jnp.int32),
        ],
    )
    return f(i0, i1)


def _scatter_body(x_hbm, i0_hbm, i1_hbm, lrA_hbm, lrB_hbm, cnt_hbm,
                  xs_hbm, posA_hbm, posB_hbm, be_hbm, cnts_hbm,
                  allcnt_v, eid_v, lr_v, posA_v, posB_v, off_ref, xbuf,
                  be_v, cntsf_v, semA, semB):
    w = _wid()
    lane = lax.iota(jnp.int32, 16)
    pltpu.sync_copy(cnt_hbm, allcnt_v)

    tot = jnp.zeros((16,), jnp.int32)
    pre = jnp.zeros((16,), jnp.int32)
    for wp in range(NW):
        row = allcnt_v[wp]
        tot = tot + row
        pre = pre + row * (wp < w).astype(jnp.int32)
    pc = (tot + (BLK - 1)) // BLK * BLK
    bexc = plsc.cumsum(pc) - pc
    off_ref[...] = bexc + pre

    for src, lr_src, pos_v, pos_dst in (
            (i0_hbm, lrA_hbm, posA_v, posA_hbm),
            (i1_hbm, lrB_hbm, posB_v, posB_hbm)):
        pltpu.sync_copy(src.at[pl.ds(w * TPW, TPW)], eid_v)
        pltpu.sync_copy(lr_src.at[pl.ds(w * TPW, TPW)], lr_v)

        def chunk(j, _):
            v = eid_v[pl.ds(j * 16, 16)]
            pos_v[pl.ds(j * 16, 16)] = (
                lr_v[pl.ds(j * 16, 16)] + plsc.load_gather(off_ref, [v]))
            return 0

        lax.fori_loop(0, CH, chunk, 0)
        pltpu.sync_copy(pos_v, pos_dst.at[pl.ds(w * TPW, TPW)])

    for j in range(CH):
        pltpu.sync_copy(x_hbm.at[pl.ds(w * TPW + j * 16, 16)], xbuf)
        a = posA_v[pl.ds(j * 16, 16)]
        b = posB_v[pl.ds(j * 16, 16)]
        cpA = pltpu.async_copy(xbuf, xs_hbm.at[a], semA)
        cpB = pltpu.async_copy(xbuf, xs_hbm.at[b], semB)
        cpA.wait()
        cpB.wait()

    @pl.when(w == 0)
    def _():
        for ch in range(NBC):
            r = (lane + 16 * ch) * BLK
            acc = jnp.zeros((16,), jnp.int32)
            for e in range(E):
                b_e = jnp.sum(jnp.where(lane == e, bexc, 0))
                acc = acc + (r >= b_e).astype(jnp.int32)
            be_v[pl.ds(ch * 16, 16)] = acc - 1
        pltpu.sync_copy(be_v, be_hbm)
        cntsf_v[...] = tot.astype(jnp.float32)
        pltpu.sync_copy(cntsf_v, cnts_hbm)


def _sc_scatter(x_flat, i0, i1, lrA, lrB, cnt_wt):
    f = pl.kernel(
        _scatter_body,
        out_type=[
            jax.ShapeDtypeStruct((NP, D), jnp.float32),
            jax.ShapeDtypeStruct((N,), jnp.int32),
            jax.ShapeDtypeStruct((N,), jnp.int32),
            jax.ShapeDtypeStruct((NBC * 16,), jnp.int32),
            jax.ShapeDtypeStruct((16,), jnp.float32),
        ],
        mesh=_mesh(),
        compiler_params=_SC_PARAMS,
        scratch_types=[
            pltpu.VMEM((NW, 16), jnp.int32),
            pltpu.VMEM((TPW,), jnp.int32),
            pltpu.VMEM((TPW,), jnp.int32),
            pltpu.VMEM((TPW,), jnp.int32),
            pltpu.VMEM((TPW,), jnp.int32),
            pltpu.VMEM((16,), jnp.int32),
            pltpu.VMEM((16, D), jnp.float32),
            pltpu.VMEM((NBC * 16,), jnp.int32),
            pltpu.VMEM((16,), jnp.float32),
            pltpu.SemaphoreType.DMA,
            pltpu.SemaphoreType.DMA,
        ],
    )
    return f(x_flat, i0, i1, lrA, lrB, cnt_wt)


def _combine_body(os_hbm, posA_hbm, posB_hbm, w0_hbm, w1_hbm, y_hbm,
                  posA_v, posB_v, w0_v, w1_v, bufsA, bufsB, ybuf,
                  semsA, semsB):
    w = _wid()
    lane = lax.iota(jnp.int32, 16)
    pltpu.sync_copy(posA_hbm.at[pl.ds(w * TPW, TPW)], posA_v)
    pltpu.sync_copy(posB_hbm.at[pl.ds(w * TPW, TPW)], posB_v)
    pltpu.sync_copy(w0_hbm.at[pl.ds(w * TPW, TPW)], w0_v)
    pltpu.sync_copy(w1_hbm.at[pl.ds(w * TPW, TPW)], w1_v)

    nck = TPW // 8

    def issue(j):
        s = j % 2
        cpA = pltpu.async_copy(
            os_hbm.at[posA_v.at[pl.ds(j * 8, 8)]], bufsA.at[s], semsA[s])
        cpB = pltpu.async_copy(
            os_hbm.at[posB_v.at[pl.ds(j * 8, 8)]], bufsB.at[s], semsB[s])
        return cpA, cpB

    pend = issue(0)
    for j in range(nck):
        s = j % 2
        pend[0].wait()
        pend[1].wait()
        if j + 1 < nck:
            pend = issue(j + 1)
        wv0 = w0_v[pl.ds((j // 2) * 16, 16)]
        wv1 = w1_v[pl.ds((j // 2) * 16, 16)]

        def row(r, _):
            rr = r + 8 * (j % 2)
            wa = jnp.sum(jnp.where(lane == rr, wv0, 0.0))
            wb = jnp.sum(jnp.where(lane == rr, wv1, 0.0))

            def col(c, _):
                ybuf[r, pl.ds(c * 16, 16)] = (
                    wa * (bufsA[s, r, pl.ds(c * 16, 16)]
                          + bufsA[s, r, pl.ds(D + c * 16, 16)])
                    + wb * (bufsB[s, r, pl.ds(c * 16, 16)]
                            + bufsB[s, r, pl.ds(D + c * 16, 16)]))
                return 0

            lax.fori_loop(0, D // 16, col, 0)
            return 0

        lax.fori_loop(0, 8, row, 0)
        pltpu.sync_copy(ybuf, y_hbm.at[pl.ds(w * TPW + j * 8, 8)])


def _sc_combine(os_rows, posA, posB, w0, w1):
    f = pl.kernel(
        _combine_body,
        out_type=jax.ShapeDtypeStruct((N, D), jnp.float32),
        mesh=_mesh(),
        compiler_params=_SC_PARAMS,
        scratch_types=[
            pltpu.VMEM((TPW,), jnp.int32),
            pltpu.VMEM((TPW,), jnp.int32),
            pltpu.VMEM((TPW,), jnp.float32),
            pltpu.VMEM((TPW,), jnp.float32),
            pltpu.VMEM((2, 8, DC * D), jnp.float32),
            pltpu.VMEM((2, 8, DC * D), jnp.float32),
            pltpu.VMEM((8, D), jnp.float32),
            [pltpu.SemaphoreType.DMA, pltpu.SemaphoreType.DMA],
            [pltpu.SemaphoreType.DMA, pltpu.SemaphoreType.DMA],
        ],
    )
    return f(os_rows, posA, posB, w0, w1)


def _ffn_body(be_ref, xs_ref, w1_ref, b1_ref, w2_ref, b2_ref, out_ref):
    c = pl.program_id(0)
    h = jnp.dot(xs_ref[...].astype(jnp.bfloat16),
                w1_ref[0].astype(jnp.bfloat16),
                preferred_element_type=jnp.float32)
    h = jnp.maximum(h + b1_ref[0], 0.0)
    o = jnp.dot(h.astype(jnp.bfloat16), w2_ref[0].astype(jnp.bfloat16),
                preferred_element_type=jnp.float32)
    out_ref[...] = o + jnp.where(c == 0, 1.0, 0.0) * b2_ref[0]


def _expert_ffn(block_expert, xs, W1, b1, W2, b2):
    grid_spec = pltpu.PrefetchScalarGridSpec(
        num_scalar_prefetch=1,
        grid=(DC, NB),
        in_specs=[
            pl.BlockSpec((BLK, D), lambda c, i, be: (i, 0)),
            pl.BlockSpec((1, D, DFC), lambda c, i, be: (be[i], 0, c)),
            pl.BlockSpec((1, 1, DFC), lambda c, i, be: (be[i], 0, c)),
            pl.BlockSpec((1, DFC, D), lambda c, i, be: (be[i], c, 0)),
            pl.BlockSpec((1, 1, D), lambda c, i, be: (be[i], 0, 0)),
        ],
        out_specs=pl.BlockSpec((BLK, D), lambda c, i, be: (i, c)),
    )
    return pl.pallas_call(
        _ffn_body,
        grid_spec=grid_spec,
        out_shape=jax.ShapeDtypeStruct((NP, DC * D), jnp.float32),
        compiler_params=pltpu.CompilerParams(
            dimension_semantics=("arbitrary", "arbitrary"),
        ),
    )(block_expert, xs, W1, b1.reshape(E, 1, DFF), W2, b2.reshape(E, 1, D))


def kernel(x, Wr, W1, b1, W2, b2):
    x_flat = x.reshape(N, D)
    i0, i1, w0, w1 = _route(x_flat, Wr)

    lrA, lrB, cnt_wt = _sc_count(i0, i1)

    xs, posA, posB, be_map, cnts16 = _sc_scatter(x_flat, i0, i1, lrA, lrB, cnt_wt)

    os_rows = _expert_ffn(be_map, xs, W1, b1, W2, b2)

    y = _sc_combine(os_rows, posA, posB, w0, w1)

    usage_counts = cnts16[:E]
    usage_fraction = usage_counts / jnp.float32(A)
    zero = jnp.zeros((), dtype=x.dtype)
    return (y.reshape(B, T, D), usage_counts, usage_fraction, zero)


# speedup vs baseline: 1.0465x; 1.0465x over previous
import functools

import jax
import jax.numpy as jnp
from jax import lax
from jax.experimental import pallas as pl
from jax.experimental.pallas import tpu as pltpu
from jax.experimental.pallas import tpu_sc as plsc

B, T, D = 2, 2048, 1024
DFF = 4096
E = 8
K = 2
N = B * T
A = N * K
BLK = 512
NB = A // BLK + E
NP = NB * BLK
NBC = (NB + 15) // 16
DFC = 2048
DC = DFF // DFC

NW = 32
TPW = N // NW
CH = TPW // 16

_mesh = functools.partial(
    plsc.VectorSubcoreMesh, core_axis_name="c", subcore_axis_name="s")
_SC_PARAMS = pltpu.CompilerParams(needs_layout_passes=False)


def _wid():
    return lax.axis_index("s") * 2 + lax.axis_index("c")


def _router_body(x_ref, wr_ref, out_ref):
    logits = jnp.dot(x_ref[...], wr_ref[...], preferred_element_type=jnp.float32)
    lane = lax.broadcasted_iota(jnp.int32, logits.shape, 1)
    neg = jnp.float32(-1e30)
    l0 = jnp.where(lane < E, logits, neg)
    m0 = jnp.max(l0, axis=1)
    i0 = jnp.min(jnp.where(l0 >= m0[:, None], lane, 127), axis=1)
    l1 = jnp.where(lane == i0[:, None], neg, l0)
    m1 = jnp.max(l1, axis=1)
    i1 = jnp.min(jnp.where(l1 >= m1[:, None], lane, 127), axis=1)
    w0 = 1.0 / (1.0 + jnp.exp(m1 - m0))
    w1 = 1.0 - w0
    out_ref[0:1, :] = i0.astype(jnp.float32)[None, :]
    out_ref[1:2, :] = i1.astype(jnp.float32)[None, :]
    out_ref[2:3, :] = w0[None, :]
    out_ref[3:4, :] = w1[None, :]


def _route(x_flat, Wr):
    wr_pad = jnp.pad(Wr, ((0, 0), (0, 128 - E)))
    r = pl.pallas_call(
        _router_body,
        out_shape=jax.ShapeDtypeStruct((8, N), jnp.float32),
    )(x_flat, wr_pad)
    i0 = r[0].astype(jnp.int32)
    i1 = r[1].astype(jnp.int32)
    return i0, i1, r[2], r[3]


def _count_body(i0_hbm, i1_hbm, lrA_hbm, lrB_hbm, cnt_hbm, eid_v, lr_v, cnt_ref):
    w = _wid()
    lane = lax.iota(jnp.int32, 16)
    cnt_ref[...] = jnp.zeros((16,), jnp.int32)

    for half, (src, dst) in enumerate(((i0_hbm, lrA_hbm), (i1_hbm, lrB_hbm))):
        pltpu.sync_copy(src.at[pl.ds(w * TPW, TPW)], eid_v)

        def chunk(j, _):
            v = eid_v[pl.ds(j * 16, 16)]
            prior = plsc.load_gather(cnt_ref, [v])
            within = jnp.zeros((16,), jnp.int32)
            cnt_vec = cnt_ref[...]
            for e in range(E):
                m = v == e
                cs = plsc.cumsum(jnp.where(m, 1, 0).astype(jnp.int32))
                within = jnp.where(m, cs - 1, within)
                tot_e = jnp.sum(jnp.where(m, 1, 0).astype(jnp.int32))
                cnt_vec = jnp.where(lane == e, cnt_vec + tot_e, cnt_vec)
            cnt_ref[...] = cnt_vec
            lr_v[pl.ds(j * 16, 16)] = prior + within
            return 0

        lax.fori_loop(0, CH, chunk, 0)
        pltpu.sync_copy(lr_v, dst.at[pl.ds(w * TPW, TPW)])

    pltpu.sync_copy(cnt_ref, cnt_hbm.at[w])


def _sc_count(i0, i1):
    f = pl.kernel(
        _count_body,
        out_type=[
            jax.ShapeDtypeStruct((N,), jnp.int32),
            jax.ShapeDtypeStruct((N,), jnp.int32),
            jax.ShapeDtypeStruct((NW, 16), jnp.int32),
        ],
        mesh=_mesh(),
        compiler_params=_SC_PARAMS,
        scratch_types=[
            pltpu.VMEM((TPW,), jnp.int32),
            pltpu.VMEM((TPW,), jnp.int32),
            pltpu.VMEM((16,), jnp.int32),
        ],
    )
    return f(i0, i1)


def _scatter_body(x_hbm, i0_hbm, i1_hbm, lrA_hbm, lrB_hbm, cnt_hbm,
                  xs_hbm, posA_hbm, posB_hbm, be_hbm, cnts_hbm,
                  allcnt_v, eid_v, lr_v, posA_v, posB_v, off_ref, xbuf,
                  be_v, cntsf_v, semA, semB):
    w = _wid()
    lane = lax.iota(jnp.int32, 16)
    pltpu.sync_copy(cnt_hbm, allcnt_v)

    tot = jnp.zeros((16,), jnp.int32)
    pre = jnp.zeros((16,), jnp.int32)
    for wp in range(NW):
        row = allcnt_v[wp]
        tot = tot + row
        pre = pre + row * (wp < w).astype(jnp.int32)
    pc = (tot + (BLK - 1)) // BLK * BLK
    bexc = plsc.cumsum(pc) - pc
    off_ref[...] = bexc + pre

    for src, lr_src, pos_v, pos_dst in (
            (i0_hbm, lrA_hbm, posA_v, posA_hbm),
            (i1_hbm, lrB_hbm, posB_v, posB_hbm)):
        pltpu.sync_copy(src.at[pl.ds(w * TPW, TPW)], eid_v)
        pltpu.sync_copy(lr_src.at[pl.ds(w * TPW, TPW)], lr_v)

        def chunk(j, _):
            v = eid_v[pl.ds(j * 16, 16)]
            pos_v[pl.ds(j * 16, 16)] = (
                lr_v[pl.ds(j * 16, 16)] + plsc.load_gather(off_ref, [v]))
            return 0

        lax.fori_loop(0, CH, chunk, 0)
        pltpu.sync_copy(pos_v, pos_dst.at[pl.ds(w * TPW, TPW)])

    for j in range(CH):
        pltpu.sync_copy(x_hbm.at[pl.ds(w * TPW + j * 16, 16)], xbuf)
        a = posA_v[pl.ds(j * 16, 16)]
        b = posB_v[pl.ds(j * 16, 16)]
        cpA = pltpu.async_copy(xbuf, xs_hbm.at[a], semA)
        cpB = pltpu.async_copy(xbuf, xs_hbm.at[b], semB)
        cpA.wait()
        cpB.wait()

    @pl.when(w == 0)
    def _():
        for ch in range(NBC):
            r = (lane + 16 * ch) * BLK
            acc = jnp.zeros((16,), jnp.int32)
            for e in range(E):
                b_e = jnp.sum(jnp.where(lane == e, bexc, 0))
                acc = acc + (r >= b_e).astype(jnp.int32)
            be_v[pl.ds(ch * 16, 16)] = acc - 1
        pltpu.sync_copy(be_v, be_hbm)
        cntsf_v[...] = tot.astype(jnp.float32)
        pltpu.sync_copy(cntsf_v, cnts_hbm)


def _sc_scatter(x_flat, i0, i1, lrA, lrB, cnt_wt):
    f = pl.kernel(
        _scatter_body,
        out_type=[
            jax.ShapeDtypeStruct((NP, D), jnp.float32),
            jax.ShapeDtypeStruct((N,), jnp.int32),
            jax.ShapeDtypeStruct((N,), jnp.int32),
            jax.ShapeDtypeStruct((NBC * 16,), jnp.int32),
            jax.ShapeDtypeStruct((16,), jnp.float32),
        ],
        mesh=_mesh(),
        compiler_params=_SC_PARAMS,
        scratch_types=[
            pltpu.VMEM((NW, 16), jnp.int32),
            pltpu.VMEM((TPW,), jnp.int32),
            pltpu.VMEM((TPW,), jnp.int32),
            pltpu.VMEM((TPW,), jnp.int32),
            pltpu.VMEM((TPW,), jnp.int32),
            pltpu.VMEM((16,), jnp.int32),
            pltpu.VMEM((16, D), jnp.float32),
            pltpu.VMEM((NBC * 16,), jnp.int32),
            pltpu.VMEM((16,), jnp.float32),
            pltpu.SemaphoreType.DMA,
            pltpu.SemaphoreType.DMA,
        ],
    )
    return f(x_flat, i0, i1, lrA, lrB, cnt_wt)


def _combine_body(os_hbm, posA_hbm, posB_hbm, w0_hbm, w1_hbm, y_hbm,
                  posA_v, posB_v, posA2_v, posB2_v, w0_v, w1_v,
                  bufsA, bufsB, ybuf, semsA, semsB):
    w = _wid()
    lane = lax.iota(jnp.int32, 16)
    pltpu.sync_copy(posA_hbm.at[pl.ds(w * TPW, TPW)], posA_v)
    pltpu.sync_copy(posB_hbm.at[pl.ds(w * TPW, TPW)], posB_v)
    pltpu.sync_copy(w0_hbm.at[pl.ds(w * TPW, TPW)], w0_v)
    pltpu.sync_copy(w1_hbm.at[pl.ds(w * TPW, TPW)], w1_v)

    def shift(j, _):
        posA2_v[pl.ds(j * 16, 16)] = posA_v[pl.ds(j * 16, 16)] + NP
        posB2_v[pl.ds(j * 16, 16)] = posB_v[pl.ds(j * 16, 16)] + NP
        return 0

    lax.fori_loop(0, CH, shift, 0)

    nck = TPW // 8

    def issue(j):
        s = j % 2
        cpA = pltpu.async_copy(
            os_hbm.at[posA_v.at[pl.ds(j * 8, 8)]], bufsA.at[s, 0], semsA[s])
        cpB = pltpu.async_copy(
            os_hbm.at[posB_v.at[pl.ds(j * 8, 8)]], bufsB.at[s, 0], semsB[s])
        cpA2 = pltpu.async_copy(
            os_hbm.at[posA2_v.at[pl.ds(j * 8, 8)]], bufsA.at[s, 1], semsA[s])
        cpB2 = pltpu.async_copy(
            os_hbm.at[posB2_v.at[pl.ds(j * 8, 8)]], bufsB.at[s, 1], semsB[s])
        return cpA, cpB, cpA2, cpB2

    pend = issue(0)
    for j in range(nck):
        s = j % 2
        for cp in pend:
            cp.wait()
        if j + 1 < nck:
            pend = issue(j + 1)
        wv0 = w0_v[pl.ds((j // 2) * 16, 16)]
        wv1 = w1_v[pl.ds((j // 2) * 16, 16)]

        def row(r, _):
            rr = r + 8 * (j % 2)
            wa = jnp.sum(jnp.where(lane == rr, wv0, 0.0))
            wb = jnp.sum(jnp.where(lane == rr, wv1, 0.0))

            def col(c, _):
                ybuf[r, pl.ds(c * 16, 16)] = (
                    wa * (bufsA[s, 0, r, pl.ds(c * 16, 16)]
                          + bufsA[s, 1, r, pl.ds(c * 16, 16)])
                    + wb * (bufsB[s, 0, r, pl.ds(c * 16, 16)]
                            + bufsB[s, 1, r, pl.ds(c * 16, 16)]))
                return 0

            lax.fori_loop(0, D // 16, col, 0)
            return 0

        lax.fori_loop(0, 8, row, 0)
        pltpu.sync_copy(ybuf, y_hbm.at[pl.ds(w * TPW + j * 8, 8)])


def _sc_combine(os_rows, posA, posB, w0, w1):
    f = pl.kernel(
        _combine_body,
        out_type=jax.ShapeDtypeStruct((N, D), jnp.float32),
        mesh=_mesh(),
        compiler_params=_SC_PARAMS,
        scratch_types=[
            pltpu.VMEM((TPW,), jnp.int32),
            pltpu.VMEM((TPW,), jnp.int32),
            pltpu.VMEM((TPW,), jnp.int32),
            pltpu.VMEM((TPW,), jnp.int32),
            pltpu.VMEM((TPW,), jnp.float32),
            pltpu.VMEM((TPW,), jnp.float32),
            pltpu.VMEM((2, DC, 8, D), jnp.float32),
            pltpu.VMEM((2, DC, 8, D), jnp.float32),
            pltpu.VMEM((8, D), jnp.float32),
            [pltpu.SemaphoreType.DMA, pltpu.SemaphoreType.DMA],
            [pltpu.SemaphoreType.DMA, pltpu.SemaphoreType.DMA],
        ],
    )
    return f(os_rows, posA, posB, w0, w1)


def _ffn_body(be_ref, xs_ref, w1_ref, b1_ref, w2_ref, b2_ref, out_ref):
    c = pl.program_id(0)
    h = jnp.dot(xs_ref[...].astype(jnp.bfloat16),
                w1_ref[0].astype(jnp.bfloat16),
                preferred_element_type=jnp.float32)
    h = jnp.maximum(h + b1_ref[0], 0.0)
    o = jnp.dot(h.astype(jnp.bfloat16), w2_ref[0].astype(jnp.bfloat16),
                preferred_element_type=jnp.float32)
    out_ref[...] = o + jnp.where(c == 0, 1.0, 0.0) * b2_ref[0]


def _expert_ffn(block_expert, xs, W1, b1, W2, b2):
    grid_spec = pltpu.PrefetchScalarGridSpec(
        num_scalar_prefetch=1,
        grid=(DC, NB),
        in_specs=[
            pl.BlockSpec((BLK, D), lambda c, i, be: (i, 0)),
            pl.BlockSpec((1, D, DFC), lambda c, i, be: (be[i], 0, c)),
            pl.BlockSpec((1, 1, DFC), lambda c, i, be: (be[i], 0, c)),
            pl.BlockSpec((1, DFC, D), lambda c, i, be: (be[i], c, 0)),
            pl.BlockSpec((1, 1, D), lambda c, i, be: (be[i], 0, 0)),
        ],
        out_specs=pl.BlockSpec((BLK, D), lambda c, i, be: (c * NB + i, 0)),
    )
    return pl.pallas_call(
        _ffn_body,
        grid_spec=grid_spec,
        out_shape=jax.ShapeDtypeStruct((DC * NP, D), jnp.float32),
        compiler_params=pltpu.CompilerParams(
            dimension_semantics=("arbitrary", "arbitrary"),
        ),
    )(block_expert, xs, W1, b1.reshape(E, 1, DFF), W2, b2.reshape(E, 1, D))


def kernel(x, Wr, W1, b1, W2, b2):
    x_flat = x.reshape(N, D)
    i0, i1, w0, w1 = _route(x_flat, Wr)

    lrA, lrB, cnt_wt = _sc_count(i0, i1)

    xs, posA, posB, be_map, cnts16 = _sc_scatter(x_flat, i0, i1, lrA, lrB, cnt_wt)

    os_rows = _expert_ffn(be_map, xs, W1, b1, W2, b2)

    y = _sc_combine(os_rows, posA, posB, w0, w1)

    usage_counts = cnts16[:E]
    usage_fraction = usage_counts / jnp.float32(A)
    zero = jnp.zeros((), dtype=x.dtype)
    return (y.reshape(B, T, D), usage_counts, usage_fraction, zero)


# speedup vs baseline: 1.0578x; 1.0108x over previous
import functools

import jax
import jax.numpy as jnp
from jax import lax
from jax.experimental import pallas as pl
from jax.experimental.pallas import tpu as pltpu
from jax.experimental.pallas import tpu_sc as plsc

B, T, D = 2, 2048, 1024
DFF = 4096
E = 8
K = 2
N = B * T
A = N * K
BLK = 512
NB = A // BLK + E
NP = NB * BLK
NBC = (NB + 15) // 16
DFC = 2048
DC = DFF // DFC

NW = 32
TPW = N // NW
CH = TPW // 16

_mesh = functools.partial(
    plsc.VectorSubcoreMesh, core_axis_name="c", subcore_axis_name="s")
_SC_PARAMS = pltpu.CompilerParams(needs_layout_passes=False)


def _wid():
    return lax.axis_index("s") * 2 + lax.axis_index("c")


def _router_body(x_ref, wr_ref, out_ref):
    logits = jnp.dot(x_ref[...], wr_ref[...], preferred_element_type=jnp.float32)
    lane = lax.broadcasted_iota(jnp.int32, logits.shape, 1)
    neg = jnp.float32(-1e30)
    l0 = jnp.where(lane < E, logits, neg)
    m0 = jnp.max(l0, axis=1)
    i0 = jnp.min(jnp.where(l0 >= m0[:, None], lane, 127), axis=1)
    l1 = jnp.where(lane == i0[:, None], neg, l0)
    m1 = jnp.max(l1, axis=1)
    i1 = jnp.min(jnp.where(l1 >= m1[:, None], lane, 127), axis=1)
    w0 = 1.0 / (1.0 + jnp.exp(m1 - m0))
    w1 = 1.0 - w0
    out_ref[0:1, :] = i0.astype(jnp.float32)[None, :]
    out_ref[1:2, :] = i1.astype(jnp.float32)[None, :]
    out_ref[2:3, :] = w0[None, :]
    out_ref[3:4, :] = w1[None, :]


def _route(x_flat, Wr):
    wr_pad = jnp.pad(Wr, ((0, 0), (0, 128 - E)))
    r = pl.pallas_call(
        _router_body,
        out_shape=jax.ShapeDtypeStruct((8, N), jnp.float32),
    )(x_flat, wr_pad)
    i0 = r[0].astype(jnp.int32)
    i1 = r[1].astype(jnp.int32)
    return i0, i1, r[2], r[3]


def _count_body(i0_hbm, i1_hbm, lrA_hbm, lrB_hbm, cnt_hbm, eid_v, lr_v, cnt_ref):
    w = _wid()
    lane = lax.iota(jnp.int32, 16)
    cnt_ref[...] = jnp.zeros((16,), jnp.int32)

    for half, (src, dst) in enumerate(((i0_hbm, lrA_hbm), (i1_hbm, lrB_hbm))):
        pltpu.sync_copy(src.at[pl.ds(w * TPW, TPW)], eid_v)

        def chunk(j, _):
            v = eid_v[pl.ds(j * 16, 16)]
            prior = plsc.load_gather(cnt_ref, [v])
            within = jnp.zeros((16,), jnp.int32)
            cnt_vec = cnt_ref[...]
            for e in range(E):
                m = v == e
                cs = plsc.cumsum(jnp.where(m, 1, 0).astype(jnp.int32))
                within = jnp.where(m, cs - 1, within)
                tot_e = jnp.sum(jnp.where(m, 1, 0).astype(jnp.int32))
                cnt_vec = jnp.where(lane == e, cnt_vec + tot_e, cnt_vec)
            cnt_ref[...] = cnt_vec
            lr_v[pl.ds(j * 16, 16)] = prior + within
            return 0

        lax.fori_loop(0, CH, chunk, 0)
        pltpu.sync_copy(lr_v, dst.at[pl.ds(w * TPW, TPW)])

    pltpu.sync_copy(cnt_ref, cnt_hbm.at[w])


def _sc_count(i0, i1):
    f = pl.kernel(
        _count_body,
        out_type=[
            jax.ShapeDtypeStruct((N,), jnp.int32),
            jax.ShapeDtypeStruct((N,), jnp.int32),
            jax.ShapeDtypeStruct((NW, 16), jnp.int32),
        ],
        mesh=_mesh(),
        compiler_params=_SC_PARAMS,
        scratch_types=[
            pltpu.VMEM((TPW,), jnp.int32),
            pltpu.VMEM((TPW,), jnp.int32),
            pltpu.VMEM((16,), jnp.int32),
        ],
    )
    return f(i0, i1)


def _scatter_body(x_hbm, i0_hbm, i1_hbm, lrA_hbm, lrB_hbm, cnt_hbm,
                  xs_hbm, posA_hbm, posB_hbm, be_hbm, cnts_hbm,
                  allcnt_v, eid_v, lr_v, posA_v, posB_v, off_ref, xbuf,
                  be_v, cntsf_v, semX, semA, semB):
    w = _wid()
    lane = lax.iota(jnp.int32, 16)
    pltpu.sync_copy(cnt_hbm, allcnt_v)

    tot = jnp.zeros((16,), jnp.int32)
    pre = jnp.zeros((16,), jnp.int32)
    for wp in range(NW):
        row = allcnt_v[wp]
        tot = tot + row
        pre = pre + row * (wp < w).astype(jnp.int32)
    pc = (tot + (BLK - 1)) // BLK * BLK
    bexc = plsc.cumsum(pc) - pc
    off_ref[...] = bexc + pre

    for src, lr_src, pos_v, pos_dst in (
            (i0_hbm, lrA_hbm, posA_v, posA_hbm),
            (i1_hbm, lrB_hbm, posB_v, posB_hbm)):
        pltpu.sync_copy(src.at[pl.ds(w * TPW, TPW)], eid_v)
        pltpu.sync_copy(lr_src.at[pl.ds(w * TPW, TPW)], lr_v)

        def chunk(j, _):
            v = eid_v[pl.ds(j * 16, 16)]
            pos_v[pl.ds(j * 16, 16)] = (
                lr_v[pl.ds(j * 16, 16)] + plsc.load_gather(off_ref, [v]))
            return 0

        lax.fori_loop(0, CH, chunk, 0)
        pltpu.sync_copy(pos_v, pos_dst.at[pl.ds(w * TPW, TPW)])

    def issue_in(j):
        return pltpu.async_copy(
            x_hbm.at[pl.ds(w * TPW + j * 16, 16)], xbuf.at[j % 2],
            semX[j % 2])

    pend_in = issue_in(0)
    pend_sc = [None, None]
    for j in range(CH):
        s = j % 2
        pend_in.wait()
        a = posA_v[pl.ds(j * 16, 16)]
        b = posB_v[pl.ds(j * 16, 16)]
        cpA = pltpu.async_copy(xbuf.at[s], xs_hbm.at[a], semA[s])
        cpB = pltpu.async_copy(xbuf.at[s], xs_hbm.at[b], semB[s])
        if pend_sc[1 - s] is not None:
            pend_sc[1 - s][0].wait()
            pend_sc[1 - s][1].wait()
        pend_sc[s] = (cpA, cpB)
        if j + 1 < CH:
            pend_in = issue_in(j + 1)
    pend_sc[(CH - 1) % 2][0].wait()
    pend_sc[(CH - 1) % 2][1].wait()

    @pl.when(w == 0)
    def _():
        for ch in range(NBC):
            r = (lane + 16 * ch) * BLK
            acc = jnp.zeros((16,), jnp.int32)
            for e in range(E):
                b_e = jnp.sum(jnp.where(lane == e, bexc, 0))
                acc = acc + (r >= b_e).astype(jnp.int32)
            be_v[pl.ds(ch * 16, 16)] = acc - 1
        pltpu.sync_copy(be_v, be_hbm)
        cntsf_v[...] = tot.astype(jnp.float32)
        pltpu.sync_copy(cntsf_v, cnts_hbm)


def _sc_scatter(x_flat, i0, i1, lrA, lrB, cnt_wt):
    f = pl.kernel(
        _scatter_body,
        out_type=[
            jax.ShapeDtypeStruct((NP, D), jnp.float32),
            jax.ShapeDtypeStruct((N,), jnp.int32),
            jax.ShapeDtypeStruct((N,), jnp.int32),
            jax.ShapeDtypeStruct((NBC * 16,), jnp.int32),
            jax.ShapeDtypeStruct((16,), jnp.float32),
        ],
        mesh=_mesh(),
        compiler_params=_SC_PARAMS,
        scratch_types=[
            pltpu.VMEM((NW, 16), jnp.int32),
            pltpu.VMEM((TPW,), jnp.int32),
            pltpu.VMEM((TPW,), jnp.int32),
            pltpu.VMEM((TPW,), jnp.int32),
            pltpu.VMEM((TPW,), jnp.int32),
            pltpu.VMEM((16,), jnp.int32),
            pltpu.VMEM((2, 16, D), jnp.float32),
            pltpu.VMEM((NBC * 16,), jnp.int32),
            pltpu.VMEM((16,), jnp.float32),
            [pltpu.SemaphoreType.DMA, pltpu.SemaphoreType.DMA],
            [pltpu.SemaphoreType.DMA, pltpu.SemaphoreType.DMA],
            [pltpu.SemaphoreType.DMA, pltpu.SemaphoreType.DMA],
        ],
    )
    return f(x_flat, i0, i1, lrA, lrB, cnt_wt)


def _combine_body(os_hbm, posA_hbm, posB_hbm, w0_hbm, w1_hbm, y_hbm,
                  posA_v, posB_v, posA2_v, posB2_v, w0_v, w1_v,
                  bufsA, bufsB, ybuf, semsA, semsB):
    w = _wid()
    lane = lax.iota(jnp.int32, 16)
    pltpu.sync_copy(posA_hbm.at[pl.ds(w * TPW, TPW)], posA_v)
    pltpu.sync_copy(posB_hbm.at[pl.ds(w * TPW, TPW)], posB_v)
    pltpu.sync_copy(w0_hbm.at[pl.ds(w * TPW, TPW)], w0_v)
    pltpu.sync_copy(w1_hbm.at[pl.ds(w * TPW, TPW)], w1_v)

    def shift(j, _):
        posA2_v[pl.ds(j * 16, 16)] = posA_v[pl.ds(j * 16, 16)] + NP
        posB2_v[pl.ds(j * 16, 16)] = posB_v[pl.ds(j * 16, 16)] + NP
        return 0

    lax.fori_loop(0, CH, shift, 0)

    nck = TPW // 8

    def issue(j):
        s = j % 2
        cpA = pltpu.async_copy(
            os_hbm.at[posA_v.at[pl.ds(j * 8, 8)]], bufsA.at[s, 0], semsA[s])
        cpB = pltpu.async_copy(
            os_hbm.at[posB_v.at[pl.ds(j * 8, 8)]], bufsB.at[s, 0], semsB[s])
        cpA2 = pltpu.async_copy(
            os_hbm.at[posA2_v.at[pl.ds(j * 8, 8)]], bufsA.at[s, 1], semsA[s])
        cpB2 = pltpu.async_copy(
            os_hbm.at[posB2_v.at[pl.ds(j * 8, 8)]], bufsB.at[s, 1], semsB[s])
        return cpA, cpB, cpA2, cpB2

    pend = issue(0)
    for j in range(nck):
        s = j % 2
        for cp in pend:
            cp.wait()
        if j + 1 < nck:
            pend = issue(j + 1)
        wv0 = w0_v[pl.ds((j // 2) * 16, 16)]
        wv1 = w1_v[pl.ds((j // 2) * 16, 16)]

        def row(r, _):
            rr = r + 8 * (j % 2)
            wa = jnp.sum(jnp.where(lane == rr, wv0, 0.0))
            wb = jnp.sum(jnp.where(lane == rr, wv1, 0.0))

            def col(c, _):
                ybuf[r, pl.ds(c * 16, 16)] = (
                    wa * (bufsA[s, 0, r, pl.ds(c * 16, 16)]
                          + bufsA[s, 1, r, pl.ds(c * 16, 16)])
                    + wb * (bufsB[s, 0, r, pl.ds(c * 16, 16)]
                            + bufsB[s, 1, r, pl.ds(c * 16, 16)]))
                return 0

            lax.fori_loop(0, D // 16, col, 0)
            return 0

        lax.fori_loop(0, 8, row, 0)
        pltpu.sync_copy(ybuf, y_hbm.at[pl.ds(w * TPW + j * 8, 8)])


def _sc_combine(os_rows, posA, posB, w0, w1):
    f = pl.kernel(
        _combine_body,
        out_type=jax.ShapeDtypeStruct((N, D), jnp.float32),
        mesh=_mesh(),
        compiler_params=_SC_PARAMS,
        scratch_types=[
            pltpu.VMEM((TPW,), jnp.int32),
            pltpu.VMEM((TPW,), jnp.int32),
            pltpu.VMEM((TPW,), jnp.int32),
            pltpu.VMEM((TPW,), jnp.int32),
            pltpu.VMEM((TPW,), jnp.float32),
            pltpu.VMEM((TPW,), jnp.float32),
            pltpu.VMEM((2, DC, 8, D), jnp.float32),
            pltpu.VMEM((2, DC, 8, D), jnp.float32),
            pltpu.VMEM((8, D), jnp.float32),
            [pltpu.SemaphoreType.DMA, pltpu.SemaphoreType.DMA],
            [pltpu.SemaphoreType.DMA, pltpu.SemaphoreType.DMA],
        ],
    )
    return f(os_rows, posA, posB, w0, w1)


def _ffn_body(be_ref, xs_ref, w1_ref, b1_ref, w2_ref, b2_ref, out_ref):
    c = pl.program_id(0)
    h = jnp.dot(xs_ref[...].astype(jnp.bfloat16),
                w1_ref[0].astype(jnp.bfloat16),
                preferred_element_type=jnp.float32)
    h = jnp.maximum(h + b1_ref[0], 0.0)
    o = jnp.dot(h.astype(jnp.bfloat16), w2_ref[0].astype(jnp.bfloat16),
                preferred_element_type=jnp.float32)
    out_ref[...] = o + jnp.where(c == 0, 1.0, 0.0) * b2_ref[0]


def _expert_ffn(block_expert, xs, W1, b1, W2, b2):
    grid_spec = pltpu.PrefetchScalarGridSpec(
        num_scalar_prefetch=1,
        grid=(DC, NB),
        in_specs=[
            pl.BlockSpec((BLK, D), lambda c, i, be: (i, 0)),
            pl.BlockSpec((1, D, DFC), lambda c, i, be: (be[i], 0, c)),
            pl.BlockSpec((1, 1, DFC), lambda c, i, be: (be[i], 0, c)),
            pl.BlockSpec((1, DFC, D), lambda c, i, be: (be[i], c, 0)),
            pl.BlockSpec((1, 1, D), lambda c, i, be: (be[i], 0, 0)),
        ],
        out_specs=pl.BlockSpec((BLK, D), lambda c, i, be: (c * NB + i, 0)),
    )
    return pl.pallas_call(
        _ffn_body,
        grid_spec=grid_spec,
        out_shape=jax.ShapeDtypeStruct((DC * NP, D), jnp.float32),
        compiler_params=pltpu.CompilerParams(
            dimension_semantics=("arbitrary", "arbitrary"),
        ),
    )(block_expert, xs, W1, b1.reshape(E, 1, DFF), W2, b2.reshape(E, 1, D))


def kernel(x, Wr, W1, b1, W2, b2):
    x_flat = x.reshape(N, D)
    i0, i1, w0, w1 = _route(x_flat, Wr)

    lrA, lrB, cnt_wt = _sc_count(i0, i1)

    xs, posA, posB, be_map, cnts16 = _sc_scatter(x_flat, i0, i1, lrA, lrB, cnt_wt)

    os_rows = _expert_ffn(be_map, xs, W1, b1, W2, b2)

    y = _sc_combine(os_rows, posA, posB, w0, w1)

    usage_counts = cnts16[:E]
    usage_fraction = usage_counts / jnp.float32(A)
    zero = jnp.zeros((), dtype=x.dtype)
    return (y.reshape(B, T, D), usage_counts, usage_fraction, zero)
